# Initial kernel scaffold; baseline (speedup 1.0000x reference)
import jax
import jax.numpy as jnp
from jax.experimental import pallas as pl


def kernel(x):
    raise NotImplementedError("write your pallas kernel here")


# speedup vs baseline: 3.9145x; 3.9145x over previous
import jax
import jax.numpy as jnp
from jax.experimental import pallas as pl

_K = 64
_MAX_ITER = 50


def _kmeans_body(x_ref, p_out_ref, idx_out_ref):
    x = x_ref[...]
    n = x.shape[0]

    x_norm = x / (jnp.sqrt(jnp.sum(x * x, axis=-1, keepdims=True)) + 1e-7)
    ones_col = jnp.ones((n, 1), dtype=jnp.float32)
    lane_iota = jax.lax.broadcasted_iota(jnp.int32, (1, _K), 1)

    def cond(state):
        _, _, i, done = state
        return jnp.logical_and(i < _MAX_ITER, jnp.logical_not(done))

    def body(state):
        p, _, i, _ = state
        p_n = p / (jnp.sqrt(jnp.sum(p * p, axis=-1, keepdims=True)) + 1e-7)
        sim = jax.lax.dot_general(
            x_norm, p_n, (((1,), (1,)), ((), ())),
            preferred_element_type=jnp.float32)
        m = jnp.max(sim, axis=-1, keepdims=True)
        idx_new = jnp.min(
            jnp.where(sim == m, lane_iota, _K), axis=-1, keepdims=True
        ).astype(jnp.int32)
        one_hot = (lane_iota == idx_new).astype(jnp.float32)
        sums = jax.lax.dot_general(
            one_hot, x, (((0,), (0,)), ((), ())),
            preferred_element_type=jnp.float32)
        counts = jax.lax.dot_general(
            one_hot, ones_col, (((0,), (0,)), ((), ())),
            preferred_element_type=jnp.float32)
        p_new = sums / (counts + 1e-6)
        variation = jnp.mean((p_new - p) ** 2)
        return (p_new, idx_new, i + 1, variation < 1e-4)

    p0 = x[:_K]
    idx0 = jnp.zeros((n, 1), dtype=jnp.int32)
    p_fin, idx_fin, _, _ = jax.lax.while_loop(
        cond, body, (p0, idx0, jnp.int32(0), jnp.bool_(False)))

    p_out_ref[...] = p_fin
    idx_out_ref[...] = idx_fin


def kernel(x):
    n, d = x.shape
    p, idx = pl.pallas_call(
        _kmeans_body,
        out_shape=(
            jax.ShapeDtypeStruct((_K, d), jnp.float32),
            jax.ShapeDtypeStruct((n, 1), jnp.int32),
        ),
    )(x)
    return (p, idx.reshape(n))


# speedup vs baseline: 13.1158x; 3.3506x over previous
import jax
import jax.numpy as jnp
from jax.experimental import pallas as pl

_K = 64
_MAX_ITER = 50


def _kmeans_body(x_ref, p_out_ref, idx_out_ref):
    x = x_ref[...]
    n = x.shape[0]

    x_norm = x / (jnp.sqrt(jnp.sum(x * x, axis=-1, keepdims=True)) + 1e-7)
    sub_iota = jax.lax.broadcasted_iota(jnp.int32, (_K, 1), 0)

    def cond(state):
        _, _, i, done = state
        return jnp.logical_and(i < _MAX_ITER, jnp.logical_not(done))

    def body(state):
        p, _, i, _ = state
        p_n = p / (jnp.sqrt(jnp.sum(p * p, axis=-1, keepdims=True)) + 1e-7)
        sim = jax.lax.dot_general(
            p_n, x_norm, (((1,), (1,)), ((), ())),
            preferred_element_type=jnp.float32)
        m = jnp.max(sim, axis=0, keepdims=True)
        idx_new = jnp.min(
            jnp.where(sim == m, sub_iota, _K), axis=0, keepdims=True
        ).astype(jnp.int32)
        one_hot = (sub_iota == idx_new).astype(jnp.float32)
        sums = jax.lax.dot_general(
            one_hot, x, (((1,), (0,)), ((), ())),
            preferred_element_type=jnp.float32)
        counts = jnp.sum(one_hot, axis=1, keepdims=True)
        p_new = sums / (counts + 1e-6)
        variation = jnp.mean((p_new - p) ** 2)
        return (p_new, idx_new, i + 1, variation < 1e-4)

    p0 = x[:_K]
    idx0 = jnp.zeros((1, n), dtype=jnp.int32)
    p_fin, idx_fin, _, _ = jax.lax.while_loop(
        cond, body, (p0, idx0, jnp.int32(0), jnp.bool_(False)))

    p_out_ref[...] = p_fin
    idx_out_ref[...] = idx_fin


def kernel(x):
    n, d = x.shape
    p, idx = pl.pallas_call(
        _kmeans_body,
        out_shape=(
            jax.ShapeDtypeStruct((_K, d), jnp.float32),
            jax.ShapeDtypeStruct((1, n), jnp.int32),
        ),
    )(x)
    return (p, idx.reshape(n))
